# Initial kernel scaffold; baseline (speedup 1.0000x reference)
import jax
import jax.numpy as jnp
from jax.experimental import pallas as pl


def kernel(x, topk_indices, rpe, distances, Wq, Wk, Wv, Wout, b_out, log_sigma):
    raise NotImplementedError("write your pallas kernel here")


# speedup vs baseline: 1.1020x; 1.1020x over previous
import functools

import jax
import jax.numpy as jnp
from jax import lax
from jax.experimental import pallas as pl
from jax.experimental.pallas import tpu as pltpu
from jax.experimental.pallas import tpu_sc as plsc

H = 12
DH = 64
DIM = 768
PE_DIM = 64
INNER = H * DH
HP = 128

_HIGH = lax.Precision.HIGHEST


def _proj_body(x_ref, w_ref, q_ref, kv_ref):
    y = jnp.dot(x_ref[...], w_ref[...], preferred_element_type=jnp.float32,
                precision=_HIGH)
    q_ref[...] = y[:, :INNER]
    kv_ref[...] = y[:, INNER:]


def _projection(x2, w_all):
    L = x2.shape[0]
    bl = 256
    return pl.pallas_call(
        _proj_body,
        grid=(L // bl,),
        in_specs=[
            pl.BlockSpec((bl, DIM), lambda i: (i, 0)),
            pl.BlockSpec((DIM, 3 * INNER), lambda i: (0, 0)),
        ],
        out_specs=[
            pl.BlockSpec((bl, INNER), lambda i: (i, 0)),
            pl.BlockSpec((bl, 2 * INNER), lambda i: (i, 0)),
        ],
        out_shape=[
            jax.ShapeDtypeStruct((L, INNER), jnp.float32),
            jax.ShapeDtypeStruct((L, 2 * INNER), jnp.float32),
        ],
    )(x2, w_all)


def _sc_gather(kv, idx_exp, chunks):
    ni = idx_exp.shape[0]
    d = kv.shape[1] // chunks
    kv = kv.reshape(kv.shape[0] * chunks, d)
    win = 128
    mesh = plsc.VectorSubcoreMesh(core_axis_name="c", subcore_axis_name="s")

    @functools.partial(
        pl.kernel,
        out_type=jax.ShapeDtypeStruct((ni, d), jnp.float32),
        mesh=mesh,
    )
    def gather_kernel(kv_hbm, i_hbm, o_hbm):
        def body(i_vmem, o_vmem):
            pltpu.sync_copy(kv_hbm.at[i_vmem.at[0]], o_vmem)

        pltpu.emit_pipeline(
            body,
            grid=(ni // win,),
            in_specs=[pl.BlockSpec((1, win), lambda i: (0, i))],
            out_specs=[pl.BlockSpec((win, d), lambda i: (i, 0))],
            core_axis_name=("c", "s"),
            dimension_semantics=(pltpu.PARALLEL,),
        )(i_hbm, o_hbm)

    return gather_kernel(kv, idx_exp.reshape(1, ni))


def _attn_body(q_ref, g_ref, rpe_ref, dist_ref, wkpe_ref, wvpe_ref,
               sel_ref, sel2_ref, wout_ref, bout_ref, sig2_ref, o_ref):
    bl = q_ref.shape[0]
    k = dist_ref.shape[1]

    g = g_ref[...]
    rpe = rpe_ref[...]
    kf = g[:, :INNER] + jnp.dot(rpe, wkpe_ref[...], precision=_HIGH)
    vf = g[:, INNER:] + jnp.dot(rpe, wvpe_ref[...], precision=_HIGH)

    q3 = q_ref[...].reshape(bl, 1, INNER)
    p = (q3 * kf.reshape(bl, k, INNER)).reshape(bl * k, INNER)
    s = jnp.dot(p, sel_ref[...], precision=_HIGH).reshape(bl, k, HP)

    dist = dist_ref[...]
    gb = -(dist * dist).reshape(bl, k, 1) * sig2_ref[...].reshape(1, 1, HP)
    s = s + gb
    m = jnp.max(s, axis=1, keepdims=True)
    e = jnp.exp(s - m)
    a = e / jnp.sum(e, axis=1, keepdims=True)

    ax = jnp.dot(a.reshape(bl * k, HP), sel2_ref[...], precision=_HIGH)
    ov = (ax * vf).reshape(bl, k, INNER).sum(axis=1)
    o_ref[...] = (jnp.dot(ov, wout_ref[...], precision=_HIGH)
                  + bout_ref[...])


def _attention(q, g, rpe2, dist, wkpe, wvpe, sel, sel2, wout, bout2, sig2):
    L = q.shape[0]
    k = dist.shape[1]
    bl = 64
    return pl.pallas_call(
        _attn_body,
        grid=(L // bl,),
        in_specs=[
            pl.BlockSpec((bl, INNER), lambda i: (i, 0)),
            pl.BlockSpec((bl * k, 2 * INNER), lambda i: (i, 0)),
            pl.BlockSpec((bl * k, PE_DIM), lambda i: (i, 0)),
            pl.BlockSpec((bl, k), lambda i: (i, 0)),
            pl.BlockSpec((PE_DIM, INNER), lambda i: (0, 0)),
            pl.BlockSpec((PE_DIM, INNER), lambda i: (0, 0)),
            pl.BlockSpec((INNER, HP), lambda i: (0, 0)),
            pl.BlockSpec((HP, INNER), lambda i: (0, 0)),
            pl.BlockSpec((INNER, DIM), lambda i: (0, 0)),
            pl.BlockSpec((1, DIM), lambda i: (0, 0)),
            pl.BlockSpec((1, HP), lambda i: (0, 0)),
        ],
        out_specs=pl.BlockSpec((bl, DIM), lambda i: (i, 0)),
        out_shape=jax.ShapeDtypeStruct((L, DIM), jnp.float32),
    )(q, g, rpe2, dist, wkpe, wvpe, sel, sel2, wout, bout2, sig2)


def kernel(x, topk_indices, rpe, distances, Wq, Wk, Wv, Wout, b_out,
           log_sigma):
    B, L, D = x.shape
    K = topk_indices.shape[-1]
    scale = DH ** (-0.5)

    chunks = 4
    x2 = x.reshape(L, D)
    idx_flat = topk_indices.reshape(L * K).astype(jnp.int32)
    idx_exp = (chunks * idx_flat[:, None]
               + jnp.arange(chunks, dtype=jnp.int32)[None, :]).reshape(-1)
    rpe2 = rpe.reshape(L * K, PE_DIM)
    dist = distances.reshape(L, K)

    w_all = jnp.concatenate([Wq, Wk[:DIM], Wv[:DIM]], axis=1)

    head_of_col = jnp.arange(INNER, dtype=jnp.int32) // DH
    hp_ids = jnp.arange(HP, dtype=jnp.int32)
    sel = (head_of_col[:, None] == hp_ids[None, :]).astype(jnp.float32) * scale
    sel2 = (hp_ids[:, None] == head_of_col[None, :]).astype(jnp.float32)

    inv2sig2 = 0.5 * jnp.exp(-2.0 * log_sigma)
    sig2 = jnp.concatenate(
        [inv2sig2, jnp.ones((HP - H,), jnp.float32)]).reshape(1, HP)
    bout2 = b_out.reshape(1, DIM)

    q, kv = _projection(x2, w_all)
    g = _sc_gather(kv, idx_exp, chunks).reshape(L * K, 2 * INNER)
    out = _attention(q, g, rpe2, dist, Wk[DIM:], Wv[DIM:],
                     sel, sel2, Wout, bout2, sig2)
    return out.reshape(B, L, DIM)


# speedup vs baseline: 2.5081x; 2.2759x over previous
import functools

import jax
import jax.numpy as jnp
from jax import lax
from jax.experimental import pallas as pl
from jax.experimental.pallas import tpu as pltpu
from jax.experimental.pallas import tpu_sc as plsc

H = 12
DH = 64
DIM = 768
PE_DIM = 64
INNER = H * DH
HP = 128

_HIGH = lax.Precision.HIGHEST
_HI16 = -65536


def _proj_body(x_ref, w_ref, q_ref, kv_ref):
    y = jnp.dot(x_ref[...], w_ref[...], preferred_element_type=jnp.float32,
                precision=_HIGH)
    q_ref[...] = y[:, :INNER]
    kb = lax.bitcast_convert_type(y[:, INNER:2 * INNER], jnp.int32)
    vb = lax.bitcast_convert_type(y[:, 2 * INNER:], jnp.int32)
    kb = (kb + 0x8000) & _HI16
    vb = ((vb + 0x8000) >> 16) & 0xFFFF
    kv_ref[...] = kb | vb


def _projection(x2, w_all):
    L = x2.shape[0]
    bl = 256
    return pl.pallas_call(
        _proj_body,
        grid=(L // bl,),
        in_specs=[
            pl.BlockSpec((bl, DIM), lambda i: (i, 0)),
            pl.BlockSpec((DIM, 3 * INNER), lambda i: (0, 0)),
        ],
        out_specs=[
            pl.BlockSpec((bl, INNER), lambda i: (i, 0)),
            pl.BlockSpec((bl, INNER), lambda i: (i, 0)),
        ],
        out_shape=[
            jax.ShapeDtypeStruct((L, INNER), jnp.float32),
            jax.ShapeDtypeStruct((L, INNER), jnp.int32),
        ],
    )(x2, w_all)


def _sc_gather(kv, idx_exp, chunks):
    ni = idx_exp.shape[0]
    d = kv.shape[1] // chunks
    kv = kv.reshape(kv.shape[0] * chunks, d)
    win = 128
    mesh = plsc.VectorSubcoreMesh(core_axis_name="c", subcore_axis_name="s")

    @functools.partial(
        pl.kernel,
        out_type=jax.ShapeDtypeStruct((ni, d), jnp.int32),
        mesh=mesh,
    )
    def gather_kernel(kv_hbm, i_hbm, o_hbm):
        def body(i_vmem, o_vmem):
            pltpu.sync_copy(kv_hbm.at[i_vmem.at[0]], o_vmem)

        pltpu.emit_pipeline(
            body,
            grid=(ni // win,),
            in_specs=[pl.BlockSpec((1, win), lambda i: (0, i))],
            out_specs=[pl.BlockSpec((win, d), lambda i: (i, 0))],
            core_axis_name=("c", "s"),
            dimension_semantics=(pltpu.PARALLEL,),
        )(i_hbm, o_hbm)

    return gather_kernel(kv, idx_exp.reshape(1, ni))


def _attn_body(q_ref, g_ref, rpe_ref, dist_ref, wkpe_ref, wvpe_ref,
               sel_ref, sel2_ref, wout_ref, bout_ref, sig2_ref, o_ref):
    bl = q_ref.shape[0]
    k = dist_ref.shape[1]

    g = g_ref[...]
    kg = lax.bitcast_convert_type(g & _HI16, jnp.float32)
    vg = lax.bitcast_convert_type(g << 16, jnp.float32)
    rpe = rpe_ref[...]
    kf = kg + jnp.dot(rpe, wkpe_ref[...], preferred_element_type=jnp.float32)
    vf = vg + jnp.dot(rpe, wvpe_ref[...], preferred_element_type=jnp.float32)

    q3 = q_ref[...].reshape(bl, 1, INNER)
    p = (q3 * kf.reshape(bl, k, INNER)).reshape(bl * k, INNER)
    s = jnp.dot(p.astype(jnp.bfloat16), sel_ref[...],
                preferred_element_type=jnp.float32).reshape(bl, k, HP)

    dist = dist_ref[...]
    gb = -(dist * dist).reshape(bl, k, 1) * sig2_ref[...].reshape(1, 1, HP)
    s = s + gb
    m = jnp.max(s, axis=1, keepdims=True)
    e = jnp.exp(s - m)
    a = e / jnp.sum(e, axis=1, keepdims=True)

    ax = jnp.dot(a.reshape(bl * k, HP).astype(jnp.bfloat16), sel2_ref[...],
                 preferred_element_type=jnp.float32)
    ov = (ax * vf).reshape(bl, k, INNER).sum(axis=1)
    o_ref[...] = (jnp.dot(ov.astype(jnp.bfloat16), wout_ref[...],
                          preferred_element_type=jnp.float32)
                  + bout_ref[...])


def _attention(q, g, rpe2, dist, wkpe, wvpe, sel, sel2, wout, bout2, sig2):
    L = q.shape[0]
    k = dist.shape[1]
    bl = 64
    return pl.pallas_call(
        _attn_body,
        grid=(L // bl,),
        in_specs=[
            pl.BlockSpec((bl, INNER), lambda i: (i, 0)),
            pl.BlockSpec((bl * k, INNER), lambda i: (i, 0)),
            pl.BlockSpec((bl * k, PE_DIM), lambda i: (i, 0)),
            pl.BlockSpec((bl, k), lambda i: (i, 0)),
            pl.BlockSpec((PE_DIM, INNER), lambda i: (0, 0)),
            pl.BlockSpec((PE_DIM, INNER), lambda i: (0, 0)),
            pl.BlockSpec((INNER, HP), lambda i: (0, 0)),
            pl.BlockSpec((HP, INNER), lambda i: (0, 0)),
            pl.BlockSpec((INNER, DIM), lambda i: (0, 0)),
            pl.BlockSpec((1, DIM), lambda i: (0, 0)),
            pl.BlockSpec((1, HP), lambda i: (0, 0)),
        ],
        out_specs=pl.BlockSpec((bl, DIM), lambda i: (i, 0)),
        out_shape=jax.ShapeDtypeStruct((L, DIM), jnp.float32),
    )(q, g, rpe2, dist, wkpe, wvpe, sel, sel2, wout, bout2, sig2)


def kernel(x, topk_indices, rpe, distances, Wq, Wk, Wv, Wout, b_out,
           log_sigma):
    B, L, D = x.shape
    K = topk_indices.shape[-1]
    scale = DH ** (-0.5)

    chunks = 2
    x2 = x.reshape(L, D)
    idx_flat = topk_indices.reshape(L * K).astype(jnp.int32)
    idx_exp = (chunks * idx_flat[:, None]
               + jnp.arange(chunks, dtype=jnp.int32)[None, :]).reshape(-1)
    rpe2 = rpe.reshape(L * K, PE_DIM).astype(jnp.bfloat16)
    dist = distances.reshape(L, K)

    w_all = jnp.concatenate([Wq, Wk[:DIM], Wv[:DIM]], axis=1)

    head_of_col = jnp.arange(INNER, dtype=jnp.int32) // DH
    hp_ids = jnp.arange(HP, dtype=jnp.int32)
    sel = ((head_of_col[:, None] == hp_ids[None, :])
           .astype(jnp.bfloat16) * jnp.bfloat16(scale))
    sel2 = (hp_ids[:, None] == head_of_col[None, :]).astype(jnp.bfloat16)

    inv2sig2 = 0.5 * jnp.exp(-2.0 * log_sigma)
    sig2 = jnp.concatenate(
        [inv2sig2, jnp.ones((HP - H,), jnp.float32)]).reshape(1, HP)
    bout2 = b_out.reshape(1, DIM)

    q, kv = _projection(x2, w_all)
    g = _sc_gather(kv, idx_exp, chunks).reshape(L * K, INNER)
    out = _attention(q, g, rpe2, dist,
                     Wk[DIM:].astype(jnp.bfloat16),
                     Wv[DIM:].astype(jnp.bfloat16),
                     sel, sel2, Wout.astype(jnp.bfloat16), bout2, sig2)
    return out.reshape(B, L, DIM)


# speedup vs baseline: 4.6332x; 1.8473x over previous
import functools

import jax
import jax.numpy as jnp
from jax import lax
from jax.experimental import pallas as pl
from jax.experimental.pallas import tpu as pltpu
from jax.experimental.pallas import tpu_sc as plsc

H = 12
DH = 64
DIM = 768
PE_DIM = 64
INNER = H * DH
HP = 128

_HIGH = lax.Precision.HIGHEST
_HI16 = -65536


def _proj_body(x_ref, w_ref, q_ref, kv_ref):
    y = jnp.dot(x_ref[...], w_ref[...], preferred_element_type=jnp.float32)
    q_ref[...] = y[:, :INNER]
    kb = lax.bitcast_convert_type(y[:, INNER:2 * INNER], jnp.int32)
    vb = lax.bitcast_convert_type(y[:, 2 * INNER:], jnp.int32)
    kb = (kb + 0x8000) & _HI16
    vb = ((vb + 0x8000) >> 16) & 0xFFFF
    kv_ref[...] = kb | vb


def _projection(x2, w_all):
    L = x2.shape[0]
    bl = 256
    return pl.pallas_call(
        _proj_body,
        grid=(L // bl,),
        in_specs=[
            pl.BlockSpec((bl, DIM), lambda i: (i, 0)),
            pl.BlockSpec((DIM, 3 * INNER), lambda i: (0, 0)),
        ],
        out_specs=[
            pl.BlockSpec((bl, INNER), lambda i: (i, 0)),
            pl.BlockSpec((bl, INNER), lambda i: (i, 0)),
        ],
        out_shape=[
            jax.ShapeDtypeStruct((L, INNER), jnp.float32),
            jax.ShapeDtypeStruct((L, INNER), jnp.int32),
        ],
    )(x2, w_all)


def _sc_gather(kv, idx_flat):
    ni = idx_flat.shape[0]
    d = kv.shape[1]
    nw = 32
    per_w = ni // nw
    chunk = 64
    nck = per_w // chunk
    mesh = plsc.VectorSubcoreMesh(core_axis_name="c", subcore_axis_name="s")

    @functools.partial(
        pl.kernel,
        out_type=jax.ShapeDtypeStruct((ni, d), jnp.int32),
        mesh=mesh,
        scratch_types=[
            pltpu.VMEM((per_w,), jnp.int32),
            pltpu.VMEM((chunk, d), jnp.int32),
            pltpu.VMEM((chunk, d), jnp.int32),
            pltpu.SemaphoreType.DMA,
            pltpu.SemaphoreType.DMA,
            pltpu.SemaphoreType.DMA,
            pltpu.SemaphoreType.DMA,
        ],
    )
    def gather_kernel(kv_hbm, i_hbm, o_hbm, idx_v, buf0, buf1,
                      gsem0, gsem1, osem0, osem1):
        wid = lax.axis_index("s") * 2 + lax.axis_index("c")
        base = wid * per_w
        pltpu.sync_copy(i_hbm.at[pl.ds(base, per_w)], idx_v)
        bufs = (buf0, buf1)
        gsems = (gsem0, gsem1)
        osems = (osem0, osem1)
        cp_g = [None, None]
        cp_o = [None, None]
        cp_g[0] = pltpu.async_copy(
            kv_hbm.at[idx_v.at[pl.ds(0, chunk)]], buf0, gsem0)
        for c in range(nck):
            b = c & 1
            nb = 1 - b
            if c + 1 < nck:
                if cp_o[nb] is not None:
                    cp_o[nb].wait()
                cp_g[nb] = pltpu.async_copy(
                    kv_hbm.at[idx_v.at[pl.ds((c + 1) * chunk, chunk)]],
                    bufs[nb], gsems[nb])
            cp_g[b].wait()
            cp_o[b] = pltpu.async_copy(
                bufs[b], o_hbm.at[pl.ds(base + c * chunk, chunk)], osems[b])
        cp_o[0].wait()
        cp_o[1].wait()

    return gather_kernel(kv, idx_flat)


def _attn_body(q_ref, g_ref, rpe_ref, dist_ref, wkpe_ref, wvpe_ref,
               sel_ref, sel2_ref, wout_ref, bout_ref, sig2_ref, o_ref):
    bl = q_ref.shape[0]
    k = dist_ref.shape[1]

    g = g_ref[...]
    kg = lax.bitcast_convert_type(g & _HI16, jnp.float32)
    vg = lax.bitcast_convert_type(g << 16, jnp.float32)
    rpe = rpe_ref[...].astype(jnp.bfloat16).reshape(bl * k, PE_DIM)
    kf = kg + jnp.dot(rpe, wkpe_ref[...], preferred_element_type=jnp.float32)
    vf = vg + jnp.dot(rpe, wvpe_ref[...], preferred_element_type=jnp.float32)

    q3 = q_ref[...].reshape(bl, 1, INNER)
    p = (q3 * kf.reshape(bl, k, INNER)).reshape(bl * k, INNER)
    s = jnp.dot(p.astype(jnp.bfloat16), sel_ref[...],
                preferred_element_type=jnp.float32).reshape(bl, k, HP)

    dist = dist_ref[...]
    gb = -(dist * dist).reshape(bl, k, 1) * sig2_ref[...].reshape(1, 1, HP)
    s = s + gb
    m = jnp.max(s, axis=1, keepdims=True)
    e = jnp.exp(s - m)
    a = e / jnp.sum(e, axis=1, keepdims=True)

    ax = jnp.dot(a.reshape(bl * k, HP).astype(jnp.bfloat16), sel2_ref[...],
                 preferred_element_type=jnp.float32)
    ov = (ax * vf).reshape(bl, k, INNER).sum(axis=1)
    o_ref[...] = (jnp.dot(ov.astype(jnp.bfloat16), wout_ref[...],
                          preferred_element_type=jnp.float32)
                  + bout_ref[...])


def _attention(q, g, rpe2, dist, wkpe, wvpe, sel, sel2, wout, bout2, sig2):
    L = q.shape[0]
    k = dist.shape[1]
    bl = 64
    return pl.pallas_call(
        _attn_body,
        grid=(L // bl,),
        in_specs=[
            pl.BlockSpec((bl, INNER), lambda i: (i, 0)),
            pl.BlockSpec((bl * k, INNER), lambda i: (i, 0)),
            pl.BlockSpec((bl, k, PE_DIM), lambda i: (i, 0, 0)),
            pl.BlockSpec((bl, k), lambda i: (i, 0)),
            pl.BlockSpec((PE_DIM, INNER), lambda i: (0, 0)),
            pl.BlockSpec((PE_DIM, INNER), lambda i: (0, 0)),
            pl.BlockSpec((INNER, HP), lambda i: (0, 0)),
            pl.BlockSpec((HP, INNER), lambda i: (0, 0)),
            pl.BlockSpec((INNER, DIM), lambda i: (0, 0)),
            pl.BlockSpec((1, DIM), lambda i: (0, 0)),
            pl.BlockSpec((1, HP), lambda i: (0, 0)),
        ],
        out_specs=pl.BlockSpec((bl, DIM), lambda i: (i, 0)),
        out_shape=jax.ShapeDtypeStruct((L, DIM), jnp.float32),
    )(q, g, rpe2, dist, wkpe, wvpe, sel, sel2, wout, bout2, sig2)


def kernel(x, topk_indices, rpe, distances, Wq, Wk, Wv, Wout, b_out,
           log_sigma):
    B, L, D = x.shape
    K = topk_indices.shape[-1]
    scale = DH ** (-0.5)

    x2 = x.reshape(L, D).astype(jnp.bfloat16)
    idx_flat = topk_indices.reshape(L * K).astype(jnp.int32)
    rpe3 = rpe.reshape(L, K, PE_DIM)
    dist = distances.reshape(L, K)

    w_all = jnp.concatenate([Wq, Wk[:DIM], Wv[:DIM]],
                            axis=1).astype(jnp.bfloat16)

    head_of_col = jnp.arange(INNER, dtype=jnp.int32) // DH
    hp_ids = jnp.arange(HP, dtype=jnp.int32)
    sel = ((head_of_col[:, None] == hp_ids[None, :])
           .astype(jnp.bfloat16) * jnp.bfloat16(scale))
    sel2 = (hp_ids[:, None] == head_of_col[None, :]).astype(jnp.bfloat16)

    inv2sig2 = 0.5 * jnp.exp(-2.0 * log_sigma)
    sig2 = jnp.concatenate(
        [inv2sig2, jnp.ones((HP - H,), jnp.float32)]).reshape(1, HP)
    bout2 = b_out.reshape(1, DIM)

    q, kv = _projection(x2, w_all)
    g = _sc_gather(kv, idx_flat)
    out = _attention(q, g, rpe3, dist,
                     Wk[DIM:].astype(jnp.bfloat16),
                     Wv[DIM:].astype(jnp.bfloat16),
                     sel, sel2, Wout.astype(jnp.bfloat16), bout2, sig2)
    return out.reshape(B, L, DIM)
